# Initial kernel scaffold; baseline (speedup 1.0000x reference)
import jax
import jax.numpy as jnp
from jax.experimental import pallas as pl


def kernel(x, tables, bias):
    raise NotImplementedError("write your pallas kernel here")


# speedup vs baseline: 6.4200x; 6.4200x over previous
import functools

import numpy as np
import jax
import jax.numpy as jnp
from jax import lax
from jax.experimental import pallas as pl
from jax.experimental.pallas import tpu as pltpu
from jax.experimental.pallas import tpu_sc as plsc

F = 26
V = 1000
D = 128
B = 4096

NW = 32
ROWS = B * F
RPW = ROWS // NW
CH = 208
NCH = RPW // CH
GROUPS = CH // F

_FOFF = np.asarray((np.arange(CH) % F) * V, dtype=np.int32)


def _body(x_hbm, foff_hbm, tab_hbm, bias_hbm, out_hbm,
          xb, rows, foff_v, bias_v, gsem):
    cid = lax.axis_index("c")
    sid = lax.axis_index("s")
    wid = sid * 2 + cid
    base = wid * RPW

    pltpu.sync_copy(bias_hbm, bias_v)
    pltpu.sync_copy(foff_hbm, foff_v)

    def chunk(c, carry):
        rbase = base + c * CH
        pltpu.sync_copy(x_hbm.at[pl.ds(rbase, CH)], xb)
        for i in range(CH // 16):
            sl = pl.ds(i * 16, 16)
            xb[sl] = xb[sl] + foff_v[sl]
        pltpu.async_copy(tab_hbm.at[xb], rows, gsem).wait()

        def rec(g, cc):
            rb = g * F
            for f in range(F):
                for t in range(D // 16):
                    sl = pl.ds(t * 16, 16)
                    rows[rb + f, sl] = rows[rb + f, sl] + bias_v[f, sl]
            return cc

        lax.fori_loop(0, GROUPS, rec, 0)
        pltpu.sync_copy(rows, out_hbm.at[pl.ds(rbase, CH)])
        return carry

    lax.fori_loop(0, NCH, chunk, 0)


def kernel(x, tables, bias):
    x_flat = x.reshape(ROWS).astype(jnp.int32)
    tab = tables.reshape(F * V, D)
    foff = jnp.asarray(_FOFF)

    mesh = plsc.VectorSubcoreMesh(core_axis_name="c", subcore_axis_name="s")
    run = pl.kernel(
        _body,
        out_type=jax.ShapeDtypeStruct((ROWS, D), jnp.float32),
        mesh=mesh,
        scratch_types=[
            pltpu.VMEM((CH,), jnp.int32),
            pltpu.VMEM((CH, D), jnp.float32),
            pltpu.VMEM((CH,), jnp.int32),
            pltpu.VMEM((F, D), jnp.float32),
            pltpu.SemaphoreType.DMA,
        ],
    )
    out = run(x_flat, foff, tab, bias)
    return out.reshape(B, F, D)


# speedup vs baseline: 7.2604x; 1.1309x over previous
import numpy as np
import jax
import jax.numpy as jnp
from jax import lax
from jax.experimental import pallas as pl
from jax.experimental.pallas import tpu as pltpu
from jax.experimental.pallas import tpu_sc as plsc

F = 26
V = 1000
D = 128
B = 4096

NW = 32
ROWS = B * F
RPW = ROWS // NW
CH = 208
NCH = RPW // CH
GROUPS = CH // F

_FOFF = np.asarray((np.arange(CH) % F) * V, dtype=np.int32)


def _body(x_hbm, foff_hbm, tab_hbm, bias_hbm, out_hbm,
          xb0, xb1, gb0, gb1, sb0, sb1, foff_v, bias_v,
          gsem0, gsem1, ssem0, ssem1):
    wid = lax.axis_index("s") * 2 + lax.axis_index("c")
    base = wid * RPW

    XB = (xb0, xb1)
    GB = (gb0, gb1)
    SB = (sb0, sb1)
    GS = (gsem0, gsem1)
    SS = (ssem0, ssem1)

    pltpu.sync_copy(bias_hbm, bias_v)
    pltpu.sync_copy(foff_hbm, foff_v)

    def fetch(c, p):
        rbase = base + c * CH
        pltpu.sync_copy(x_hbm.at[pl.ds(rbase, CH)], XB[p])
        for i in range(CH // 16):
            sl = pl.ds(i * 16, 16)
            XB[p][sl] = XB[p][sl] + foff_v[sl]
        pltpu.async_copy(tab_hbm.at[XB[p]], GB[p], GS[p])

    def wait_gather(p):
        pltpu.make_async_copy(tab_hbm.at[XB[p]], GB[p], GS[p]).wait()

    def bias_add(p):
        def rec(g, cc):
            rb = g * F
            for f in range(F):
                for t in range(D // 16):
                    sl = pl.ds(t * 16, 16)
                    SB[p][rb + f, sl] = GB[p][rb + f, sl] + bias_v[f, sl]
            return cc
        lax.fori_loop(0, GROUPS, rec, 0)

    def store(c, p):
        rbase = base + c * CH
        pltpu.async_copy(SB[p], out_hbm.at[pl.ds(rbase, CH)], SS[p])

    def wait_store(p):
        pltpu.make_async_copy(SB[p], out_hbm.at[pl.ds(base, CH)], SS[p]).wait()

    fetch(0, 0)
    fetch(1, 1)

    for p in (0, 1):
        wait_gather(p)
        bias_add(p)
        store(p, p)
        fetch(p + 2, p)

    def main(k, carry):
        for p in (0, 1):
            c = 2 * k + p
            wait_gather(p)
            wait_store(p)
            bias_add(p)
            store(c, p)
            fetch(c + 2, p)
        return carry

    lax.fori_loop(1, NCH // 2 - 1, main, 0)

    for p in (0, 1):
        wait_gather(p)
        wait_store(p)
        bias_add(p)
        store(14 + p, p)

    for p in (0, 1):
        wait_store(p)


def kernel(x, tables, bias):
    x_flat = x.reshape(ROWS).astype(jnp.int32)
    tab = tables.reshape(F * V, D)
    foff = jnp.asarray(_FOFF)

    mesh = plsc.VectorSubcoreMesh(core_axis_name="c", subcore_axis_name="s")
    run = pl.kernel(
        _body,
        out_type=jax.ShapeDtypeStruct((ROWS, D), jnp.float32),
        mesh=mesh,
        scratch_types=[
            pltpu.VMEM((CH,), jnp.int32),
            pltpu.VMEM((CH,), jnp.int32),
            pltpu.VMEM((CH, D), jnp.float32),
            pltpu.VMEM((CH, D), jnp.float32),
            pltpu.VMEM((CH, D), jnp.float32),
            pltpu.VMEM((CH, D), jnp.float32),
            pltpu.VMEM((CH,), jnp.int32),
            pltpu.VMEM((F, D), jnp.float32),
            pltpu.SemaphoreType.DMA,
            pltpu.SemaphoreType.DMA,
            pltpu.SemaphoreType.DMA,
            pltpu.SemaphoreType.DMA,
        ],
    )
    out = run(x_flat, foff, tab, bias)
    return out.reshape(B, F, D)


# speedup vs baseline: 10.2259x; 1.4085x over previous
import numpy as np
import jax
import jax.numpy as jnp
from jax import lax
from jax.experimental import pallas as pl
from jax.experimental.pallas import tpu as pltpu
from jax.experimental.pallas import tpu_sc as plsc

F = 26
V = 1000
D = 128
B = 4096

NW = 32
ROWS = B * F
RPW = ROWS // NW
CH = 128
NCH = RPW // CH
NBUF = 6
DIST = 4

_FOFF = np.asarray((np.arange(ROWS) % F) * V, dtype=np.int32)


def _fuse_body(tab_ref, bias_ref, out_ref):
    out_ref[...] = tab_ref[...] + bias_ref[...]


def _fuse(tables, bias):
    return pl.pallas_call(
        _fuse_body,
        grid=(F,),
        in_specs=[
            pl.BlockSpec((1, V, D), lambda f: (f, 0, 0)),
            pl.BlockSpec((1, 1, D), lambda f: (f, 0, 0)),
        ],
        out_specs=pl.BlockSpec((1, V, D), lambda f: (f, 0, 0)),
        out_shape=jax.ShapeDtypeStruct((F, V, D), jnp.float32),
    )(tables, bias.reshape(F, 1, D))


def _gather_body(x_hbm, foff_hbm, tab_hbm, out_hbm,
                 xb0, xb1, xb2, xb3, xb4, xb5,
                 fb0, fb1, fb2, fb3, fb4, fb5,
                 gb0, gb1, gb2, gb3, gb4, gb5,
                 gs0, gs1, gs2, gs3, gs4, gs5,
                 ss0, ss1, ss2, ss3, ss4, ss5):
    wid = lax.axis_index("s") * 2 + lax.axis_index("c")
    base = wid * RPW

    XB = (xb0, xb1, xb2, xb3, xb4, xb5)
    FB = (fb0, fb1, fb2, fb3, fb4, fb5)
    GB = (gb0, gb1, gb2, gb3, gb4, gb5)
    GS = (gs0, gs1, gs2, gs3, gs4, gs5)
    SS = (ss0, ss1, ss2, ss3, ss4, ss5)

    def wait_store(q):
        pltpu.make_async_copy(GB[q], out_hbm.at[pl.ds(base, CH)], SS[q]).wait()

    def fetch(c, q, wait):
        if wait:
            wait_store(q)
        rbase = base + c * CH
        pltpu.sync_copy(x_hbm.at[pl.ds(rbase, CH)], XB[q])
        pltpu.sync_copy(foff_hbm.at[pl.ds(rbase, CH)], FB[q])
        for i in range(CH // 16):
            sl = pl.ds(i * 16, 16)
            XB[q][sl] = XB[q][sl] + FB[q][sl]
        pltpu.async_copy(tab_hbm.at[XB[q]], GB[q], GS[q])

    def body(c, p):
        pltpu.make_async_copy(tab_hbm.at[XB[p]], GB[p], GS[p]).wait()
        pltpu.async_copy(GB[p], out_hbm.at[pl.ds(base + c * CH, CH)], SS[p])

    for c in range(DIST):
        fetch(c, c % NBUF, wait=False)

    for c in range(NBUF):
        body(c, c % NBUF)
        fetch(c + DIST, (c + DIST) % NBUF, wait=(c + DIST >= NBUF))

    def main(k, carry):
        for p in range(NBUF):
            c = NBUF * k + p
            body(c, p)
            fetch(c + DIST, (p + DIST) % NBUF, wait=True)
        return carry

    lax.fori_loop(1, 3, main, 0)

    for c in range(3 * NBUF, NCH):
        body(c, c % NBUF)
        if c + DIST < NCH:
            fetch(c + DIST, (c + DIST) % NBUF, wait=True)

    for q in range(NBUF):
        wait_store(q)


def kernel(x, tables, bias):
    x_flat = x.reshape(ROWS).astype(jnp.int32)
    fused = _fuse(tables, bias).reshape(F * V, D)
    foff = jnp.asarray(_FOFF)

    mesh = plsc.VectorSubcoreMesh(core_axis_name="c", subcore_axis_name="s")
    run = pl.kernel(
        _gather_body,
        out_type=jax.ShapeDtypeStruct((ROWS, D), jnp.float32),
        mesh=mesh,
        scratch_types=(
            [pltpu.VMEM((CH,), jnp.int32) for _ in range(NBUF)]
            + [pltpu.VMEM((CH,), jnp.int32) for _ in range(NBUF)]
            + [pltpu.VMEM((CH, D), jnp.float32) for _ in range(NBUF)]
            + [pltpu.SemaphoreType.DMA for _ in range(NBUF)]
            + [pltpu.SemaphoreType.DMA for _ in range(NBUF)]
        ),
    )
    out = run(x_flat, foff, fused)
    return out.reshape(B, F, D)
